# Initial kernel scaffold; baseline (speedup 1.0000x reference)
import jax
import jax.numpy as jnp
from jax.experimental import pallas as pl


def kernel(x, edge_index, node_types, Wnt, bnt, W1, b1, Wc1, as1, ad1, bc1, Wc2, as2, ad2, bc2, Wc3, as3, ad3, bc3):
    raise NotImplementedError("write your pallas kernel here")


# speedup vs baseline: 1.0009x; 1.0009x over previous
import functools

import jax
import jax.numpy as jnp
from jax.experimental import pallas as pl
from jax.experimental.pallas import tpu as pltpu

N = 10000
D = 128


def _pre_body(x_ref, oh_ref, Wnt_ref, bnt_ref, W1_ref, b1_ref, o_ref):
    x = x_ref[...]
    oh = oh_ref[...]
    t = jnp.dot(oh[:, :5], bnt_ref[...], preferred_element_type=jnp.float32)
    for k in range(5):
        t = t + jnp.dot(x, Wnt_ref[k], preferred_element_type=jnp.float32) * oh[:, k : k + 1]
    h = jnp.maximum(jnp.dot(t, W1_ref[...], preferred_element_type=jnp.float32) + b1_ref[...], 0.0)
    o_ref[...] = h


def _pre_stage(x, node_types, Wnt, bnt, W1, b1):
    oh = jax.nn.one_hot(node_types, 8, dtype=jnp.float32)
    B = 1000
    grid = (N // B,)
    return pl.pallas_call(
        _pre_body,
        grid=grid,
        in_specs=[
            pl.BlockSpec((B, D), lambda i: (i, 0)),
            pl.BlockSpec((B, 8), lambda i: (i, 0)),
            pl.BlockSpec((5, D, 64), lambda i: (0, 0, 0)),
            pl.BlockSpec((5, 64), lambda i: (0, 0)),
            pl.BlockSpec((64, 64), lambda i: (0, 0)),
            pl.BlockSpec((1, 64), lambda i: (0, 0)),
        ],
        out_specs=pl.BlockSpec((B, 64), lambda i: (i, 0)),
        out_shape=jax.ShapeDtypeStruct((N, 64), jnp.float32),
    )(x, oh, Wnt, bnt, W1, b1.reshape(1, 64))


def _gat(x, src, dst, W, a_s, a_d, b, H, C):
    n = x.shape[0]
    h = (x @ W).reshape(n, H, C)
    al = (h * a_s).sum(-1)
    ar = (h * a_d).sum(-1)
    e = jax.nn.leaky_relu(al[src] + ar[dst], 0.2)
    m = jax.ops.segment_max(e, dst, num_segments=n)
    ex = jnp.exp(e - m[dst])
    den = jax.ops.segment_sum(ex, dst, num_segments=n)
    alpha = ex / (den[dst] + 1e-16)
    out = jax.ops.segment_sum(h[src] * alpha[:, :, None], dst, num_segments=n)
    return out.reshape(n, H * C) + b


def kernel(x, edge_index, node_types, Wnt, bnt, W1, b1, Wc1, as1, ad1, bc1, Wc2, as2, ad2, bc2, Wc3, as3, ad3, bc3):
    n = x.shape[0]
    loop = jnp.arange(n, dtype=edge_index.dtype)
    src = jnp.concatenate([edge_index[0], loop])
    dst = jnp.concatenate([edge_index[1], loop])
    h = _pre_stage(x, node_types, Wnt, bnt, W1, b1)
    h = jax.nn.relu(_gat(h, src, dst, Wc1, as1, ad1, bc1, 2, 32))
    h = jax.nn.relu(_gat(h, src, dst, Wc2, as2, ad2, bc2, 2, 64))
    h = _gat(h, src, dst, Wc3, as3, ad3, bc3, 1, D)
    return h


# speedup vs baseline: 53.3563x; 53.3069x over previous
import functools

import jax
import jax.numpy as jnp
from jax import lax
from jax.experimental import pallas as pl
from jax.experimental.pallas import tpu as pltpu
from jax.experimental.pallas import tpu_sc as plsc

N = 10000
D = 128
N_PAD = 10240
E_PAD = 651264
EPW2 = E_PAD // 32
EPW1 = E_PAD // 16
SC1 = EPW1 // 8
K = 128

_MESH = plsc.VectorSubcoreMesh(core_axis_name="c", subcore_axis_name="s",
                               num_cores=2, num_subcores=16)
_SC_PARAMS = pltpu.CompilerParams(needs_layout_passes=False)


def _pre1_body(x_ref, oh_ref, Wnt_ref, bnt_ref, W1_ref, b1_ref, Wc_ref,
               As_ref, Ad_ref, hw_ref, al_ref, ar_ref):
    x = x_ref[...]
    oh = oh_ref[...]
    t = jnp.dot(oh[:, :5], bnt_ref[...], preferred_element_type=jnp.float32)
    for k in range(5):
        t = t + jnp.dot(x, Wnt_ref[k], preferred_element_type=jnp.float32) * oh[:, k:k + 1]
    h1 = jnp.maximum(jnp.dot(t, W1_ref[...], preferred_element_type=jnp.float32) + b1_ref[...], 0.0)
    hw = jnp.dot(h1, Wc_ref[...], preferred_element_type=jnp.float32)
    hw_ref[...] = jnp.concatenate([hw, jnp.zeros_like(hw)], axis=1)
    al_ref[...] = jnp.dot(hw, As_ref[...], preferred_element_type=jnp.float32)
    ar_ref[...] = jnp.dot(hw, Ad_ref[...], preferred_element_type=jnp.float32)


def _pre1(x, node_types, Wnt, bnt, W1, b1, Wc, As, Ad):
    oh = jax.nn.one_hot(node_types, 8, dtype=jnp.float32)
    B = 1000
    full = lambda *s: pl.BlockSpec(s, lambda i: (0,) * len(s))
    return pl.pallas_call(
        _pre1_body,
        grid=(N // B,),
        in_specs=[
            pl.BlockSpec((B, D), lambda i: (i, 0)),
            pl.BlockSpec((B, 8), lambda i: (i, 0)),
            full(5, D, 64), full(5, 64), full(64, 64), full(1, 64),
            full(64, 64), full(64, 8), full(64, 8),
        ],
        out_specs=[
            pl.BlockSpec((B, 128), lambda i: (i, 0)),
            pl.BlockSpec((B, 8), lambda i: (i, 0)),
            pl.BlockSpec((B, 8), lambda i: (i, 0)),
        ],
        out_shape=[
            jax.ShapeDtypeStruct((N, 128), jnp.float32),
            jax.ShapeDtypeStruct((N, 8), jnp.float32),
            jax.ShapeDtypeStruct((N, 8), jnp.float32),
        ],
    )(x, oh, Wnt, bnt, W1, b1.reshape(1, 64), Wc, As, Ad)


def _mid_body(p0_ref, p1_ref, bc_ref, Wc_ref, As_ref, Ad_ref,
              hw_ref, al_ref, ar_ref):
    act = jnp.maximum(p0_ref[...] + p1_ref[...] + bc_ref[...], 0.0)
    hw = jnp.dot(act, Wc_ref[...], preferred_element_type=jnp.float32)
    hw_ref[...] = hw
    al_ref[...] = jnp.dot(hw, As_ref[...], preferred_element_type=jnp.float32)
    ar_ref[...] = jnp.dot(hw, Ad_ref[...], preferred_element_type=jnp.float32)


def _mid(p0, p1, bc, Wc, As, Ad):
    F = p0.shape[1]
    HC = Wc.shape[1]
    B = 1000
    full = lambda *s: pl.BlockSpec(s, lambda i: (0,) * len(s))
    return pl.pallas_call(
        _mid_body,
        grid=(N // B,),
        in_specs=[
            pl.BlockSpec((B, F), lambda i: (i, 0)),
            pl.BlockSpec((B, F), lambda i: (i, 0)),
            full(1, F), full(F, HC), full(HC, 8), full(HC, 8),
        ],
        out_specs=[
            pl.BlockSpec((B, HC), lambda i: (i, 0)),
            pl.BlockSpec((B, 8), lambda i: (i, 0)),
            pl.BlockSpec((B, 8), lambda i: (i, 0)),
        ],
        out_shape=[
            jax.ShapeDtypeStruct((N, HC), jnp.float32),
            jax.ShapeDtypeStruct((N, 8), jnp.float32),
            jax.ShapeDtypeStruct((N, 8), jnp.float32),
        ],
    )(p0[:N], p1[:N], bc.reshape(1, F), Wc, As, Ad)


def _fin_body(p0_ref, p1_ref, bc_ref, o_ref):
    o_ref[...] = p0_ref[...] + p1_ref[...] + bc_ref[...]


def _fin(p0, p1, bc):
    F = p0.shape[1]
    B = 1000
    return pl.pallas_call(
        _fin_body,
        grid=(N // B,),
        in_specs=[
            pl.BlockSpec((B, F), lambda i: (i, 0)),
            pl.BlockSpec((B, F), lambda i: (i, 0)),
            pl.BlockSpec((1, F), lambda i: (0, 0)),
        ],
        out_specs=pl.BlockSpec((B, F), lambda i: (i, 0)),
        out_shape=jax.ShapeDtypeStruct((N, F), jnp.float32),
    )(p0[:N], p1[:N], bc.reshape(1, F))


def _att_body(H, src_hbm, dst_hbm, alf_hbm, arf_hbm, alpha_hbm,
              al_v, ar_v, den_v, es_v, ed_v, src_v, dst_v, alpha_v,
              merge_v, msum_v, den_slots, den_sum):
    HN = H * N_PAD
    SEG = HN // 16
    cid = lax.axis_index("c")
    sid = lax.axis_index("s")
    wid = sid * 2 + cid

    pltpu.sync_copy(alf_hbm, al_v)
    pltpu.sync_copy(arf_hbm, ar_v)
    zero16 = jnp.zeros((16,), jnp.float32)

    def _zden(i, _):
        den_v[pl.ds(i * 16, 16)] = zero16
        return _
    lax.fori_loop(0, HN // 16, _zden, None)

    for b in range(8):
        base = sid * EPW1 + b * SC1
        pltpu.sync_copy(src_hbm.at[pl.ds(base, SC1)], es_v)
        pltpu.sync_copy(dst_hbm.at[pl.ds(base, SC1)], ed_v)

        def _p1(g, _):
            s = es_v[pl.ds(g * 16, 16)]
            d = ed_v[pl.ds(g * 16, 16)]
            for h in range(H):
                als = plsc.load_gather(al_v, [s + h * N_PAD])
                ard = plsc.load_gather(ar_v, [d + h * N_PAD])
                e = als + ard
                e = jnp.where(e >= 0.0, e, 0.2 * e)
                plsc.addupdate_scatter(den_v, [d + h * N_PAD], jnp.exp(e))
            return _
        lax.fori_loop(0, SC1 // 16, _p1, None)

    plsc.subcore_barrier()

    @pl.when(sid >= 8)
    def _():
        pltpu.sync_copy(den_v, den_slots.at[sid - 8])
    plsc.subcore_barrier()

    @pl.when(sid < 8)
    def _():
        pltpu.sync_copy(den_slots.at[sid], merge_v)

        def _acc(i, _):
            den_v[pl.ds(i * 16, 16)] = (den_v[pl.ds(i * 16, 16)]
                                        + merge_v[pl.ds(i * 16, 16)])
            return _
        lax.fori_loop(0, HN // 16, _acc, None)
        pltpu.sync_copy(den_v, den_slots.at[sid])
    plsc.subcore_barrier()

    off = sid * SEG
    for kq in range(8):
        pltpu.sync_copy(den_slots.at[kq, pl.ds(off, SEG)],
                        merge_v.at[pl.ds(kq * SEG, SEG)])

    def _mrg(j, _):
        acc = merge_v[pl.ds(j * 16, 16)]
        for kq in range(1, 8):
            acc = acc + merge_v[pl.ds(kq * SEG + j * 16, 16)]
        msum_v[pl.ds(j * 16, 16)] = acc
        return _
    lax.fori_loop(0, SEG // 16, _mrg, None)
    pltpu.sync_copy(msum_v, den_sum.at[pl.ds(off, SEG)])
    plsc.subcore_barrier()
    pltpu.sync_copy(den_sum, den_v)

    def _p2(t, _):
        ebase = wid * EPW2 + t * K
        pltpu.sync_copy(src_hbm.at[pl.ds(ebase, K)], src_v)
        pltpu.sync_copy(dst_hbm.at[pl.ds(ebase, K)], dst_v)
        for g in range(K // 16):
            s = src_v[pl.ds(g * 16, 16)]
            d = dst_v[pl.ds(g * 16, 16)]
            for h in range(H):
                als = plsc.load_gather(al_v, [s + h * N_PAD])
                ard = plsc.load_gather(ar_v, [d + h * N_PAD])
                e = als + ard
                e = jnp.where(e >= 0.0, e, 0.2 * e)
                den = plsc.load_gather(den_v, [d + h * N_PAD])
                alpha_v[pl.ds(h * K + g * 16, 16)] = jnp.exp(e) / (den + 1e-16)
        for h in range(H):
            pltpu.sync_copy(alpha_v.at[pl.ds(h * K, K)],
                            alpha_hbm.at[pl.ds(h * E_PAD + ebase, K)])
        return _
    lax.fori_loop(0, EPW2 // K, _p2, None)


def _att_sc(src, dst, alf, arf, H):
    HN = H * N_PAD
    body = functools.partial(_att_body, H)
    k = pl.kernel(
        body,
        out_type=jax.ShapeDtypeStruct((H * E_PAD,), jnp.float32),
        mesh=_MESH,
        scratch_types=[
            pltpu.VMEM((HN,), jnp.float32),
            pltpu.VMEM((HN,), jnp.float32),
            pltpu.VMEM((HN,), jnp.float32),
            pltpu.VMEM((SC1,), jnp.int32),
            pltpu.VMEM((SC1,), jnp.int32),
            pltpu.VMEM((K,), jnp.int32),
            pltpu.VMEM((K,), jnp.int32),
            pltpu.VMEM((H * K,), jnp.float32),
            pltpu.VMEM((HN,), jnp.float32),
            pltpu.VMEM((HN // 16,), jnp.float32),
            pltpu.VMEM_SHARED((8, HN), jnp.float32),
            pltpu.VMEM_SHARED((HN,), jnp.float32),
        ],
        compiler_params=_SC_PARAMS,
    )
    return k(src, dst, alf, arf)


def _agg_body(H, C, src_hbm, dst_hbm, alpha_hbm, h_hbm, part_hbm,
              src_v, dst_v, alpha_v, rows_v, out_sh, sem):
    cid = lax.axis_index("c")
    sid = lax.axis_index("s")
    wid = sid * 2 + cid
    zero16 = jnp.zeros((16,), jnp.float32)

    def _zrow(i, _):
        for r in range(128 // 16):
            rows_v[i, pl.ds(r * 16, 16)] = zero16
        return _
    lax.fori_loop(0, K, _zrow, None)
    for kk in range(N_PAD // 16 // K):
        pltpu.sync_copy(rows_v, out_sh.at[pl.ds(sid * (N_PAD // 16) + kk * K, K)])
    plsc.subcore_barrier()

    def _p(t, _):
        ebase = wid * EPW2 + t * K
        pltpu.sync_copy(src_hbm.at[pl.ds(ebase, K)], src_v)
        pltpu.sync_copy(dst_hbm.at[pl.ds(ebase, K)], dst_v)
        gat = pltpu.async_copy(h_hbm.at[src_v], rows_v, sem)
        for h in range(H):
            pltpu.sync_copy(alpha_hbm.at[pl.ds(h * E_PAD + ebase, K)],
                            alpha_v.at[pl.ds(h * K, K)])
        gat.wait()

        def _scale(j, _):
            for h in range(H):
                a = plsc.load_gather(
                    alpha_v, [lax.broadcast(j + h * K, (16,)).astype(jnp.int32)])
                for r in range(C // 16):
                    col = h * C + r * 16
                    rows_v[j, pl.ds(col, 16)] = rows_v[j, pl.ds(col, 16)] * a
            return _
        lax.fori_loop(0, K, _scale, None)
        pltpu.sync_copy(rows_v, out_sh.at[dst_v], add=True)
        return _
    lax.fori_loop(0, EPW2 // K, _p, None)

    plsc.subcore_barrier()
    row0 = sid * (N_PAD // 16)
    pltpu.sync_copy(out_sh.at[pl.ds(row0, N_PAD // 16)],
                    part_hbm.at[cid, pl.ds(row0, N_PAD // 16)])


def _agg_sc(src, dst, alpha, hW, H, C):
    body = functools.partial(_agg_body, H, C)
    k = pl.kernel(
        body,
        out_type=jax.ShapeDtypeStruct((2, N_PAD, 128), jnp.float32),
        mesh=_MESH,
        scratch_types=[
            pltpu.VMEM((K,), jnp.int32),
            pltpu.VMEM((K,), jnp.int32),
            pltpu.VMEM((H * K,), jnp.float32),
            pltpu.VMEM((K, 128), jnp.float32),
            pltpu.VMEM_SHARED((N_PAD, 128), jnp.float32),
            pltpu.SemaphoreType.DMA,
        ],
        compiler_params=_SC_PARAMS,
    )
    return k(src, dst, alpha, hW)


def _gat_sc(src, dst, alf, arf, hW, H, C):
    alpha = _att_sc(src, dst, alf, arf, H)
    return _agg_sc(src, dst, alpha, hW, H, C)


def _make_A(a, H, C):
    A = jnp.zeros((H, C, 8), jnp.float32)
    for h in range(H):
        A = A.at[h, :, h].set(a[0, h])
    return A.reshape(H * C, 8)


def _flatT(v, H):
    return jnp.pad(v[:, :H].T, ((0, 0), (0, N_PAD - N))).reshape(-1)


def kernel(x, edge_index, node_types, Wnt, bnt, W1, b1, Wc1, as1, ad1, bc1,
           Wc2, as2, ad2, bc2, Wc3, as3, ad3, bc3):
    loop = jnp.arange(N, dtype=jnp.int32)
    pad = E_PAD - (edge_index.shape[1] + N)
    src = jnp.concatenate([edge_index[0], loop,
                           jnp.zeros((pad,), jnp.int32)])
    dst = jnp.concatenate([edge_index[1], loop,
                           jnp.full((pad,), N, jnp.int32)])

    hW, al, ar = _pre1(x, node_types, Wnt, bnt, W1, b1, Wc1,
                       _make_A(as1, 2, 32), _make_A(ad1, 2, 32))
    p = _gat_sc(src, dst, _flatT(al, 2), _flatT(ar, 2), hW, 2, 32)

    hW, al, ar = _mid(p[0][:, :64], p[1][:, :64], bc1, Wc2,
                      _make_A(as2, 2, 64), _make_A(ad2, 2, 64))
    p = _gat_sc(src, dst, _flatT(al, 2), _flatT(ar, 2), hW, 2, 64)

    hW, al, ar = _mid(p[0], p[1], bc2, Wc3,
                      _make_A(as3, 1, 128), _make_A(ad3, 1, 128))
    p = _gat_sc(src, dst, _flatT(al, 1), _flatT(ar, 1), hW, 1, 128)

    return _fin(p[0], p[1], bc3)
